# baseline (speedup 1.0000x reference)
import functools

import jax
import jax.numpy as jnp
from jax import lax
from jax.experimental import pallas as pl
from jax.experimental.pallas import tpu as pltpu
from jax.experimental.pallas import tpu_sc as plsc

N = 10000
D = 128
E = 320000
NC = 2
NS = 16
NW = NC * NS
C = 96
T = 216
TOTCH = NS * T
EPAD = TOTCH * C
NF = 152
NSL = 64
FAST_CORE = 0
G = 8
RING = 3 * G
NCHK1 = 108
SINK = N
ACCR = 10008
NPAD = 10240
RPW = N // NS
ZR = 125

_vmesh = functools.partial(
    plsc.VectorSubcoreMesh, core_axis_name="c", subcore_axis_name="s")


def _deg_body(dst_hbm, zeros1_hbm, degp_hbm, acc, dstv, ones, sem):
  c = lax.axis_index("c")
  s = lax.axis_index("s")
  w = c * NS + s
  pltpu.sync_copy(zeros1_hbm.at[pl.ds(s * (NPAD // NS), NPAD // NS)],
                  acc.at[pl.ds(s * (NPAD // NS), NPAD // NS)])
  @pl.loop(0, 8)
  def _(i):
    ones[pl.ds(i * 16, 16)] = jnp.ones((16,), jnp.float32)
  pltpu.async_copy(dst_hbm.at[w], dstv, sem).wait()
  plsc.subcore_barrier()

  @pl.loop(0, NCHK1)
  def _(j):
    pltpu.sync_copy(ones.at[pl.ds(0, C)], acc.at[dstv.at[j]], add=True)

  plsc.subcore_barrier()
  @pl.when(s == 0)
  def _():
    pltpu.sync_copy(acc, degp_hbm.at[c])


def _deg_partials(dst3, zeros1):
  return pl.kernel(
      _deg_body,
      out_type=jax.ShapeDtypeStruct((NC, NPAD), jnp.float32),
      mesh=_vmesh(),
      scratch_types=[
          pltpu.VMEM_SHARED((NPAD,), jnp.float32),
          pltpu.VMEM((NCHK1, C), jnp.int32),
          pltpu.VMEM((128,), jnp.float32),
          pltpu.SemaphoreType.DMA,
      ],
  )(dst3, zeros1)


def _scale_body(x_ref, w_ref, d0_ref, d1_ref, scaled_ref, dis_ref):
  deg = d0_ref[...] + d1_ref[...] + 1.0
  dis = 1.0 / jnp.sqrt(deg)
  h = jnp.dot(x_ref[...], w_ref[...], preferred_element_type=jnp.float32)
  scaled_ref[...] = h * dis
  dis_ref[...] = dis


def _matmul_scale(x, W, deg0, deg1):
  B = 400
  grid = (N // B,)
  return pl.pallas_call(
      _scale_body,
      grid=grid,
      in_specs=[
          pl.BlockSpec((B, D), lambda i: (i, 0)),
          pl.BlockSpec((D, D), lambda i: (0, 0)),
          pl.BlockSpec((B, 1), lambda i: (i, 0)),
          pl.BlockSpec((B, 1), lambda i: (i, 0)),
      ],
      out_specs=[
          pl.BlockSpec((B, D), lambda i: (i, 0)),
          pl.BlockSpec((B, 1), lambda i: (i, 0)),
      ],
      out_shape=[
          jax.ShapeDtypeStruct((N, D), jnp.float32),
          jax.ShapeDtypeStruct((N, 1), jnp.float32),
      ],
  )(x, W, deg0, deg1)


def _run_edges(scaled_hbm, src_hbm, dst2_hbm, acc, sidx, didx, rows0, rows1,
               sem0, sem1, isem, base_chunk, nch):
  base_chunk = pl.multiple_of(base_chunk, 8)
  pltpu.sync_copy(src_hbm.at[pl.ds(base_chunk * C, G * C)],
                  sidx.at[pl.ds(0, G * C)])
  pltpu.sync_copy(dst2_hbm.at[pl.ds(base_chunk, G)], didx.at[pl.ds(0, G)])
  if nch > G:
    pltpu.async_copy(src_hbm.at[pl.ds((base_chunk + G) * C, G * C)],
                     sidx.at[pl.ds(G * C, G * C)], isem)
    pltpu.async_copy(dst2_hbm.at[pl.ds(pl.multiple_of(base_chunk + G, 8), G)],
                     didx.at[pl.ds(G, G)], isem)
  pltpu.async_copy(scaled_hbm.at[sidx.at[pl.ds(0, C)]], rows0, sem0)

  @pl.loop(0, nch, step=2)
  def _(j):
    slot0 = lax.rem(j, RING)
    slot1 = lax.rem(j + 1, RING)
    slot2 = lax.rem(j + 2, RING)
    pltpu.make_async_copy(scaled_hbm.at[sidx.at[pl.ds(slot0 * C, C)]],
                          rows0, sem0).wait()
    pltpu.async_copy(scaled_hbm.at[sidx.at[pl.ds(slot1 * C, C)]],
                     rows1, sem1)
    pltpu.sync_copy(rows0, acc.at[didx.at[slot0]], add=True)
    pltpu.make_async_copy(scaled_hbm.at[sidx.at[pl.ds(slot1 * C, C)]],
                          rows1, sem1).wait()
    boundary = lax.rem(j + 2, G) == 0

    @pl.when(jnp.logical_and(boundary, j + 2 < nch))
    def _():
      pltpu.make_async_copy(src_hbm.at[pl.ds(base_chunk * C, G * C)],
                            sidx.at[pl.ds(0, G * C)], isem).wait()
      pltpu.make_async_copy(dst2_hbm.at[pl.ds(base_chunk, G)],
                            didx.at[pl.ds(0, G)], isem).wait()

    @pl.when(jnp.logical_and(boundary, j + 2 + G < nch))
    def _():
      ns = pl.multiple_of(lax.rem(j + 2 + G, RING), 8)
      nxt = pl.multiple_of(base_chunk + j + 2 + G, 8)
      pltpu.async_copy(src_hbm.at[pl.ds(nxt * C, G * C)],
                       sidx.at[pl.ds(ns * C, G * C)], isem)
      pltpu.async_copy(dst2_hbm.at[pl.ds(nxt, G)],
                       didx.at[pl.ds(ns, G)], isem)

    @pl.when(j + 2 < nch)
    def _():
      pltpu.async_copy(scaled_hbm.at[sidx.at[pl.ds(slot2 * C, C)]],
                       rows0, sem0)
    pltpu.sync_copy(rows1, acc.at[didx.at[slot1]], add=True)


def _agg_body(scaled_hbm, src_hbm, dst2_hbm, zeros2_hbm, part_hbm,
              acc, sidx, didx, rows0, rows1, sem0, sem1, isem):
  c = lax.axis_index("c")
  s = lax.axis_index("s")
  w = c * NS + s
  @pl.loop(0, RPW // ZR)
  def _(i):
    pltpu.sync_copy(zeros2_hbm, acc.at[pl.ds(s * RPW + i * ZR, ZR)])
  plsc.subcore_barrier()

  args = (scaled_hbm, src_hbm, dst2_hbm, acc, sidx, didx, rows0, rows1,
          sem0, sem1, isem)

  @pl.when(c == FAST_CORE)
  def _():
    _run_edges(*args, s * T, NF)

  @pl.when(c != FAST_CORE)
  def _():
    _run_edges(*args, s * T + NF, NSL)

  plsc.subcore_barrier()
  pltpu.sync_copy(acc.at[pl.ds(s * RPW, RPW)], part_hbm.at[w])


def _edge_aggregate(scaled, src_flat, dst2, zeros2):
  return pl.kernel(
      _agg_body,
      out_type=jax.ShapeDtypeStruct((NW, RPW, D), jnp.float32),
      mesh=_vmesh(),
      scratch_types=[
          pltpu.VMEM_SHARED((ACCR, D), jnp.float32),
          pltpu.VMEM((RING * C,), jnp.int32),
          pltpu.VMEM((RING, C), jnp.int32),
          pltpu.VMEM((C, D), jnp.float32),
          pltpu.VMEM((C, D), jnp.float32),
          pltpu.SemaphoreType.DMA,
          pltpu.SemaphoreType.DMA,
          pltpu.SemaphoreType.DMA,
      ],
  )(scaled, src_flat, dst2, zeros2)


def _ln_body(p0_ref, p1_ref, sc_ref, dis_ref, x_ref, b_ref, g_ref, bt_ref,
             out_ref):
  agg = (p0_ref[...] + p1_ref[...] + sc_ref[...]) * dis_ref[...] + b_ref[...]
  mu = jnp.mean(agg, axis=-1, keepdims=True)
  zc = agg - mu
  var = jnp.mean(zc * zc, axis=-1, keepdims=True)
  ln = zc / jnp.sqrt(var + 1e-5) * g_ref[...] + bt_ref[...]
  out_ref[...] = jnp.maximum(ln, 0.0) + x_ref[...]


def _ln_residual(p0, p1, scaled, dis, x, b, g, bt):
  B = 400
  grid = (N // B,)
  row = lambda i: (i, 0)
  return pl.pallas_call(
      _ln_body,
      grid=grid,
      in_specs=[
          pl.BlockSpec((B, D), row),
          pl.BlockSpec((B, D), row),
          pl.BlockSpec((B, D), row),
          pl.BlockSpec((B, 1), row),
          pl.BlockSpec((B, D), row),
          pl.BlockSpec((1, D), lambda i: (0, 0)),
          pl.BlockSpec((1, D), lambda i: (0, 0)),
          pl.BlockSpec((1, D), lambda i: (0, 0)),
      ],
      out_specs=pl.BlockSpec((B, D), row),
      out_shape=jax.ShapeDtypeStruct((N, D), jnp.float32),
  )(p0, p1, scaled, dis, x, b, g, bt)


def kernel(x, edge_index, W, b, ln_gamma, ln_beta):
  ei = edge_index.astype(jnp.int32)
  npad = EPAD - E
  src_flat = jnp.concatenate([ei[0], jnp.zeros((npad,), jnp.int32)])
  dst_flat = jnp.concatenate([ei[1], jnp.full((npad,), SINK, jnp.int32)])
  dst2 = dst_flat.reshape(TOTCH, C)
  dst3 = dst_flat.reshape(NW, NCHK1, C)
  zeros1 = jnp.zeros((NPAD,), jnp.float32)
  zeros2 = jnp.zeros((ZR, D), jnp.float32)

  degp = _deg_partials(dst3, zeros1)
  deg0 = degp[0, :N].reshape(N, 1)
  deg1 = degp[1, :N].reshape(N, 1)

  scaled, dis = _matmul_scale(x, W, deg0, deg1)

  parts = _edge_aggregate(scaled, src_flat, dst2, zeros2)
  p = parts.reshape(NC, N, D)

  return _ln_residual(p[0], p[1], scaled, dis, x, b.reshape(1, D),
                      ln_gamma.reshape(1, D), ln_beta.reshape(1, D))


# speedup vs baseline: 1.0100x; 1.0100x over previous
import functools

import jax
import jax.numpy as jnp
from jax import lax
from jax.experimental import pallas as pl
from jax.experimental.pallas import tpu as pltpu
from jax.experimental.pallas import tpu_sc as plsc

N = 10000
D = 128
E = 320000
NC = 2
NS = 16
NW = NC * NS
C = 96
T = 216
TOTCH = NS * T
EPAD = TOTCH * C
NF = 152
NSL = 64
FAST_CORE = 1
G = 8
RING = 3 * G
NCHK1 = 108
SINK = N
ACCR = 10008
NPAD = 10240
RPW = N // NS
ZR = 125

_vmesh = functools.partial(
    plsc.VectorSubcoreMesh, core_axis_name="c", subcore_axis_name="s")


def _deg_body(dst_hbm, zeros1_hbm, degp_hbm, acc, dstv, ones, sem):
  c = lax.axis_index("c")
  s = lax.axis_index("s")
  w = c * NS + s
  pltpu.sync_copy(zeros1_hbm.at[pl.ds(s * (NPAD // NS), NPAD // NS)],
                  acc.at[pl.ds(s * (NPAD // NS), NPAD // NS)])
  @pl.loop(0, 8)
  def _(i):
    ones[pl.ds(i * 16, 16)] = jnp.ones((16,), jnp.float32)
  pltpu.async_copy(dst_hbm.at[w], dstv, sem).wait()
  plsc.subcore_barrier()

  @pl.loop(0, NCHK1)
  def _(j):
    pltpu.sync_copy(ones.at[pl.ds(0, C)], acc.at[dstv.at[j]], add=True)

  plsc.subcore_barrier()
  @pl.when(s == 0)
  def _():
    pltpu.sync_copy(acc, degp_hbm.at[c])


def _deg_partials(dst3, zeros1):
  return pl.kernel(
      _deg_body,
      out_type=jax.ShapeDtypeStruct((NC, NPAD), jnp.float32),
      mesh=_vmesh(),
      scratch_types=[
          pltpu.VMEM_SHARED((NPAD,), jnp.float32),
          pltpu.VMEM((NCHK1, C), jnp.int32),
          pltpu.VMEM((128,), jnp.float32),
          pltpu.SemaphoreType.DMA,
      ],
  )(dst3, zeros1)


def _scale_body(x_ref, w_ref, d0_ref, d1_ref, scaled_ref, dis_ref):
  deg = d0_ref[...] + d1_ref[...] + 1.0
  dis = 1.0 / jnp.sqrt(deg)
  h = jnp.dot(x_ref[...], w_ref[...], preferred_element_type=jnp.float32)
  scaled_ref[...] = h * dis
  dis_ref[...] = dis


def _matmul_scale(x, W, deg0, deg1):
  B = 400
  grid = (N // B,)
  return pl.pallas_call(
      _scale_body,
      grid=grid,
      in_specs=[
          pl.BlockSpec((B, D), lambda i: (i, 0)),
          pl.BlockSpec((D, D), lambda i: (0, 0)),
          pl.BlockSpec((B, 1), lambda i: (i, 0)),
          pl.BlockSpec((B, 1), lambda i: (i, 0)),
      ],
      out_specs=[
          pl.BlockSpec((B, D), lambda i: (i, 0)),
          pl.BlockSpec((B, 1), lambda i: (i, 0)),
      ],
      out_shape=[
          jax.ShapeDtypeStruct((N, D), jnp.float32),
          jax.ShapeDtypeStruct((N, 1), jnp.float32),
      ],
  )(x, W, deg0, deg1)


def _run_edges(scaled_hbm, src_hbm, dst2_hbm, acc, sidx, didx, rows0, rows1,
               sem0, sem1, isem, base_chunk, nch):
  base_chunk = pl.multiple_of(base_chunk, 8)
  pltpu.sync_copy(src_hbm.at[pl.ds(base_chunk * C, G * C)],
                  sidx.at[pl.ds(0, G * C)])
  pltpu.sync_copy(dst2_hbm.at[pl.ds(base_chunk, G)], didx.at[pl.ds(0, G)])
  if nch > G:
    pltpu.async_copy(src_hbm.at[pl.ds((base_chunk + G) * C, G * C)],
                     sidx.at[pl.ds(G * C, G * C)], isem)
    pltpu.async_copy(dst2_hbm.at[pl.ds(pl.multiple_of(base_chunk + G, 8), G)],
                     didx.at[pl.ds(G, G)], isem)
  pltpu.async_copy(scaled_hbm.at[sidx.at[pl.ds(0, C)]], rows0, sem0)

  @pl.loop(0, nch, step=2)
  def _(j):
    slot0 = lax.rem(j, RING)
    slot1 = lax.rem(j + 1, RING)
    slot2 = lax.rem(j + 2, RING)
    pltpu.make_async_copy(scaled_hbm.at[sidx.at[pl.ds(slot0 * C, C)]],
                          rows0, sem0).wait()
    pltpu.async_copy(scaled_hbm.at[sidx.at[pl.ds(slot1 * C, C)]],
                     rows1, sem1)
    pltpu.sync_copy(rows0, acc.at[didx.at[slot0]], add=True)
    pltpu.make_async_copy(scaled_hbm.at[sidx.at[pl.ds(slot1 * C, C)]],
                          rows1, sem1).wait()
    boundary = lax.rem(j + 2, G) == 0

    @pl.when(jnp.logical_and(boundary, j + 2 < nch))
    def _():
      pltpu.make_async_copy(src_hbm.at[pl.ds(base_chunk * C, G * C)],
                            sidx.at[pl.ds(0, G * C)], isem).wait()
      pltpu.make_async_copy(dst2_hbm.at[pl.ds(base_chunk, G)],
                            didx.at[pl.ds(0, G)], isem).wait()

    @pl.when(jnp.logical_and(boundary, j + 2 + G < nch))
    def _():
      ns = pl.multiple_of(lax.rem(j + 2 + G, RING), 8)
      nxt = pl.multiple_of(base_chunk + j + 2 + G, 8)
      pltpu.async_copy(src_hbm.at[pl.ds(nxt * C, G * C)],
                       sidx.at[pl.ds(ns * C, G * C)], isem)
      pltpu.async_copy(dst2_hbm.at[pl.ds(nxt, G)],
                       didx.at[pl.ds(ns, G)], isem)

    @pl.when(j + 2 < nch)
    def _():
      pltpu.async_copy(scaled_hbm.at[sidx.at[pl.ds(slot2 * C, C)]],
                       rows0, sem0)
    pltpu.sync_copy(rows1, acc.at[didx.at[slot1]], add=True)


def _agg_body(scaled_hbm, src_hbm, dst2_hbm, zeros2_hbm, part_hbm,
              acc, sidx, didx, rows0, rows1, sem0, sem1, isem):
  c = lax.axis_index("c")
  s = lax.axis_index("s")
  w = c * NS + s
  @pl.loop(0, RPW // ZR)
  def _(i):
    pltpu.sync_copy(zeros2_hbm, acc.at[pl.ds(s * RPW + i * ZR, ZR)])
  plsc.subcore_barrier()

  args = (scaled_hbm, src_hbm, dst2_hbm, acc, sidx, didx, rows0, rows1,
          sem0, sem1, isem)

  @pl.when(c == FAST_CORE)
  def _():
    _run_edges(*args, s * T, NF)

  @pl.when(c != FAST_CORE)
  def _():
    _run_edges(*args, s * T + NF, NSL)

  plsc.subcore_barrier()
  pltpu.sync_copy(acc.at[pl.ds(s * RPW, RPW)], part_hbm.at[w])


def _edge_aggregate(scaled, src_flat, dst2, zeros2):
  return pl.kernel(
      _agg_body,
      out_type=jax.ShapeDtypeStruct((NW, RPW, D), jnp.float32),
      mesh=_vmesh(),
      scratch_types=[
          pltpu.VMEM_SHARED((ACCR, D), jnp.float32),
          pltpu.VMEM((RING * C,), jnp.int32),
          pltpu.VMEM((RING, C), jnp.int32),
          pltpu.VMEM((C, D), jnp.float32),
          pltpu.VMEM((C, D), jnp.float32),
          pltpu.SemaphoreType.DMA,
          pltpu.SemaphoreType.DMA,
          pltpu.SemaphoreType.DMA,
      ],
  )(scaled, src_flat, dst2, zeros2)


def _ln_body(p0_ref, p1_ref, sc_ref, dis_ref, x_ref, b_ref, g_ref, bt_ref,
             out_ref):
  agg = (p0_ref[...] + p1_ref[...] + sc_ref[...]) * dis_ref[...] + b_ref[...]
  mu = jnp.mean(agg, axis=-1, keepdims=True)
  zc = agg - mu
  var = jnp.mean(zc * zc, axis=-1, keepdims=True)
  ln = zc / jnp.sqrt(var + 1e-5) * g_ref[...] + bt_ref[...]
  out_ref[...] = jnp.maximum(ln, 0.0) + x_ref[...]


def _ln_residual(p0, p1, scaled, dis, x, b, g, bt):
  B = 400
  grid = (N // B,)
  row = lambda i: (i, 0)
  return pl.pallas_call(
      _ln_body,
      grid=grid,
      in_specs=[
          pl.BlockSpec((B, D), row),
          pl.BlockSpec((B, D), row),
          pl.BlockSpec((B, D), row),
          pl.BlockSpec((B, 1), row),
          pl.BlockSpec((B, D), row),
          pl.BlockSpec((1, D), lambda i: (0, 0)),
          pl.BlockSpec((1, D), lambda i: (0, 0)),
          pl.BlockSpec((1, D), lambda i: (0, 0)),
      ],
      out_specs=pl.BlockSpec((B, D), row),
      out_shape=jax.ShapeDtypeStruct((N, D), jnp.float32),
  )(p0, p1, scaled, dis, x, b, g, bt)


def kernel(x, edge_index, W, b, ln_gamma, ln_beta):
  ei = edge_index.astype(jnp.int32)
  npad = EPAD - E
  src_flat = jnp.concatenate([ei[0], jnp.zeros((npad,), jnp.int32)])
  dst_flat = jnp.concatenate([ei[1], jnp.full((npad,), SINK, jnp.int32)])
  dst2 = dst_flat.reshape(TOTCH, C)
  dst3 = dst_flat.reshape(NW, NCHK1, C)
  zeros1 = jnp.zeros((NPAD,), jnp.float32)
  zeros2 = jnp.zeros((ZR, D), jnp.float32)

  degp = _deg_partials(dst3, zeros1)
  deg0 = degp[0, :N].reshape(N, 1)
  deg1 = degp[1, :N].reshape(N, 1)

  scaled, dis = _matmul_scale(x, W, deg0, deg1)

  parts = _edge_aggregate(scaled, src_flat, dst2, zeros2)
  p = parts.reshape(NC, N, D)

  return _ln_residual(p[0], p[1], scaled, dis, x, b.reshape(1, D),
                      ln_gamma.reshape(1, D), ln_beta.reshape(1, D))


# speedup vs baseline: 2.6537x; 2.6274x over previous
import functools

import jax
import jax.numpy as jnp
from jax import lax
from jax.experimental import pallas as pl
from jax.experimental.pallas import tpu as pltpu
from jax.experimental.pallas import tpu_sc as plsc

N = 10000
D = 128
E = 320000
NC = 2
NS = 16
NW = NC * NS
C = 96
NCHUNK = 106
EPP = C * NCHUNK
EPAD = NW * EPP
ACCR = 10160
NPAD = 10240
RPW = N // NS
ZR = 125

_vmesh = functools.partial(
    plsc.VectorSubcoreMesh, core_axis_name="c", subcore_axis_name="s")


def _deg_body(dst_hbm, zeros1_hbm, degp_hbm, acc, dstv, ones, sem):
  c = lax.axis_index("c")
  s = lax.axis_index("s")
  w = c * NS + s
  pltpu.sync_copy(zeros1_hbm.at[pl.ds(s * (NPAD // NS), NPAD // NS)],
                  acc.at[pl.ds(s * (NPAD // NS), NPAD // NS)])
  @pl.loop(0, 8)
  def _(i):
    ones[pl.ds(i * 16, 16)] = jnp.ones((16,), jnp.float32)
  pltpu.async_copy(dst_hbm.at[w], dstv, sem).wait()
  plsc.subcore_barrier()

  @pl.loop(0, NCHUNK)
  def _(j):
    pltpu.sync_copy(ones.at[pl.ds(0, C)], acc.at[dstv.at[j]], add=True)

  plsc.subcore_barrier()
  @pl.when(s == 0)
  def _():
    pltpu.sync_copy(acc, degp_hbm.at[c])


def _deg_partials(dst3, zeros1):
  return pl.kernel(
      _deg_body,
      out_type=jax.ShapeDtypeStruct((NC, NPAD), jnp.float32),
      mesh=_vmesh(),
      scratch_types=[
          pltpu.VMEM_SHARED((NPAD,), jnp.float32),
          pltpu.VMEM((NCHUNK, C), jnp.int32),
          pltpu.VMEM((128,), jnp.float32),
          pltpu.SemaphoreType.DMA,
      ],
  )(dst3, zeros1)


def _scale_body(x_ref, w_ref, d0_ref, d1_ref, scaled_ref, dis_ref):
  deg = d0_ref[...] + d1_ref[...] + 1.0
  dis = 1.0 / jnp.sqrt(deg)
  h = jnp.dot(x_ref[...], w_ref[...], preferred_element_type=jnp.float32)
  scaled_ref[...] = h * dis
  dis_ref[...] = dis


def _matmul_scale(x, W, deg0, deg1):
  B = 2000
  grid = (N // B,)
  return pl.pallas_call(
      _scale_body,
      grid=grid,
      in_specs=[
          pl.BlockSpec((B, D), lambda i: (i, 0)),
          pl.BlockSpec((D, D), lambda i: (0, 0)),
          pl.BlockSpec((B, 1), lambda i: (i, 0)),
          pl.BlockSpec((B, 1), lambda i: (i, 0)),
      ],
      out_specs=[
          pl.BlockSpec((B, D), lambda i: (i, 0)),
          pl.BlockSpec((B, 1), lambda i: (i, 0)),
      ],
      out_shape=[
          jax.ShapeDtypeStruct((N, D), jnp.float32),
          jax.ShapeDtypeStruct((N, 1), jnp.float32),
      ],
  )(x, W, deg0, deg1)


def _agg_body(scaled_hbm, src_hbm, dst_hbm, zeros2_hbm, part_hbm,
              acc, srcv, dstv, rows0, rows1, sem0, sem1, isem):
  c = lax.axis_index("c")
  s = lax.axis_index("s")
  w = c * NS + s
  @pl.loop(0, RPW // ZR)
  def _(i):
    pltpu.sync_copy(zeros2_hbm, acc.at[pl.ds(s * RPW + i * ZR, ZR)])
  @pl.when(s < (ACCR - N) // 16)
  def _():
    pltpu.sync_copy(zeros2_hbm.at[pl.ds(0, 16)],
                    acc.at[pl.ds(N + s * 16, 16)])
  pltpu.async_copy(src_hbm.at[pl.ds(w * EPP, EPP)], srcv, isem).wait()
  pltpu.async_copy(dst_hbm.at[w], dstv, isem).wait()
  plsc.subcore_barrier()

  pltpu.async_copy(scaled_hbm.at[srcv.at[pl.ds(0, C)]], rows0, sem0)

  @pl.loop(0, NCHUNK, step=2)
  def _(j):
    pltpu.make_async_copy(
        scaled_hbm.at[srcv.at[pl.ds(j * C, C)]], rows0, sem0).wait()
    pltpu.async_copy(
        scaled_hbm.at[srcv.at[pl.ds((j + 1) * C, C)]], rows1, sem1)
    pltpu.sync_copy(rows0, acc.at[dstv.at[j]], add=True)
    pltpu.make_async_copy(
        scaled_hbm.at[srcv.at[pl.ds((j + 1) * C, C)]], rows1, sem1).wait()
    @pl.when(j + 2 < NCHUNK)
    def _():
      pltpu.async_copy(
          scaled_hbm.at[srcv.at[pl.ds((j + 2) * C, C)]], rows0, sem0)
    pltpu.sync_copy(rows1, acc.at[dstv.at[j + 1]], add=True)

  plsc.subcore_barrier()
  pltpu.sync_copy(acc.at[pl.ds(s * RPW, RPW)], part_hbm.at[w])


def _edge_aggregate(scaled, src_flat, dst3, zeros2):
  return pl.kernel(
      _agg_body,
      out_type=jax.ShapeDtypeStruct((NW, RPW, D), jnp.float32),
      mesh=_vmesh(),
      scratch_types=[
          pltpu.VMEM_SHARED((ACCR, D), jnp.float32),
          pltpu.VMEM((EPP,), jnp.int32),
          pltpu.VMEM((NCHUNK, C), jnp.int32),
          pltpu.VMEM((C, D), jnp.float32),
          pltpu.VMEM((C, D), jnp.float32),
          pltpu.SemaphoreType.DMA,
          pltpu.SemaphoreType.DMA,
          pltpu.SemaphoreType.DMA,
      ],
  )(scaled, src_flat, dst3, zeros2)


def _ln_body(p0_ref, p1_ref, sc_ref, dis_ref, x_ref, b_ref, g_ref, bt_ref,
             out_ref):
  agg = (p0_ref[...] + p1_ref[...] + sc_ref[...]) * dis_ref[...] + b_ref[...]
  mu = jnp.mean(agg, axis=-1, keepdims=True)
  zc = agg - mu
  var = jnp.mean(zc * zc, axis=-1, keepdims=True)
  ln = zc / jnp.sqrt(var + 1e-5) * g_ref[...] + bt_ref[...]
  out_ref[...] = jnp.maximum(ln, 0.0) + x_ref[...]


def _ln_residual(p0, p1, scaled, dis, x, b, g, bt):
  B = 2000
  grid = (N // B,)
  row = lambda i: (i, 0)
  return pl.pallas_call(
      _ln_body,
      grid=grid,
      in_specs=[
          pl.BlockSpec((B, D), row),
          pl.BlockSpec((B, D), row),
          pl.BlockSpec((B, D), row),
          pl.BlockSpec((B, 1), row),
          pl.BlockSpec((B, D), row),
          pl.BlockSpec((1, D), lambda i: (0, 0)),
          pl.BlockSpec((1, D), lambda i: (0, 0)),
          pl.BlockSpec((1, D), lambda i: (0, 0)),
      ],
      out_specs=pl.BlockSpec((B, D), row),
      out_shape=jax.ShapeDtypeStruct((N, D), jnp.float32),
  )(p0, p1, scaled, dis, x, b, g, bt)


def kernel(x, edge_index, W, b, ln_gamma, ln_beta):
  ei = edge_index.astype(jnp.int32)
  npad = EPAD - E
  pad_i = jnp.arange(npad, dtype=jnp.int32)
  src_flat = jnp.concatenate([ei[0], pad_i % N])
  dst_flat = jnp.concatenate([ei[1], N + pad_i % (ACCR - N)])
  dst3 = dst_flat.reshape(NW, NCHUNK, C)
  zeros1 = jnp.zeros((NPAD,), jnp.float32)
  zeros2 = jnp.zeros((ZR, D), jnp.float32)

  degp = _deg_partials(dst3, zeros1)
  deg0 = degp[0, :N].reshape(N, 1)
  deg1 = degp[1, :N].reshape(N, 1)

  scaled, dis = _matmul_scale(x, W, deg0, deg1)

  parts = _edge_aggregate(scaled, src_flat, dst3, zeros2)
  p = parts.reshape(NC, N, D)

  return _ln_residual(p[0], p[1], scaled, dis, x, b.reshape(1, D),
                      ln_gamma.reshape(1, D), ln_beta.reshape(1, D))


# speedup vs baseline: 3.1653x; 1.1928x over previous
import functools

import jax
import jax.numpy as jnp
from jax import lax
from jax.experimental import pallas as pl
from jax.experimental.pallas import tpu as pltpu
from jax.experimental.pallas import tpu_sc as plsc

N = 10000
D = 128
E = 320000
NC = 2
NS = 16
NW = NC * NS
C = 96
NCHUNK = 106
EPP = C * NCHUNK
EPAD = NW * EPP
ACCR = 10160
NPAD = 10240
SLAB = 640
LAST = 400
ZR = 160

_vmesh = functools.partial(
    plsc.VectorSubcoreMesh, core_axis_name="c", subcore_axis_name="s")


def _deg_body(dst_hbm, zeros1_hbm, degp_hbm, acc, dstv, ones, sem):
  c = lax.axis_index("c")
  s = lax.axis_index("s")
  w = c * NS + s
  pltpu.sync_copy(zeros1_hbm.at[pl.ds(s * (NPAD // NS), NPAD // NS)],
                  acc.at[pl.ds(s * (NPAD // NS), NPAD // NS)])
  @pl.loop(0, 8)
  def _(i):
    ones[pl.ds(i * 16, 16)] = jnp.ones((16,), jnp.float32)
  pltpu.async_copy(dst_hbm.at[w], dstv, sem).wait()
  plsc.subcore_barrier()

  @pl.loop(0, NCHUNK)
  def _(j):
    pltpu.sync_copy(ones.at[pl.ds(0, C)], acc.at[dstv.at[j]], add=True)

  plsc.subcore_barrier()
  @pl.when(s == 0)
  def _():
    pltpu.sync_copy(acc, degp_hbm.at[c])


def _deg_partials(dst3, zeros1):
  return pl.kernel(
      _deg_body,
      out_type=jax.ShapeDtypeStruct((NC, NPAD), jnp.float32),
      mesh=_vmesh(),
      scratch_types=[
          pltpu.VMEM_SHARED((NPAD,), jnp.float32),
          pltpu.VMEM((NCHUNK, C), jnp.int32),
          pltpu.VMEM((128,), jnp.float32),
          pltpu.SemaphoreType.DMA,
      ],
  )(dst3, zeros1)


def _mm_body(x_ref, w_ref, h_ref):
  h_ref[...] = jnp.dot(x_ref[...], w_ref[...],
                       preferred_element_type=jnp.float32)


def _matmul(x, W):
  B = 2000
  return pl.pallas_call(
      _mm_body,
      grid=(N // B,),
      in_specs=[
          pl.BlockSpec((B, D), lambda i: (i, 0)),
          pl.BlockSpec((D, D), lambda i: (0, 0)),
      ],
      out_specs=pl.BlockSpec((B, D), lambda i: (i, 0)),
      out_shape=jax.ShapeDtypeStruct((N, D), jnp.float32),
  )(x, W)


def _scale_body(h_ref, d0_ref, d1_ref, scaled_ref, dis_ref):
  deg = d0_ref[...] + d1_ref[...] + 1.0
  dis = 1.0 / jnp.sqrt(deg)
  scaled_ref[...] = h_ref[...] * dis
  dis_ref[...] = dis


def _row_scale(h, deg0, deg1):
  B = 2000
  return pl.pallas_call(
      _scale_body,
      grid=(N // B,),
      in_specs=[
          pl.BlockSpec((B, D), lambda i: (i, 0)),
          pl.BlockSpec((B, 1), lambda i: (i, 0)),
          pl.BlockSpec((B, 1), lambda i: (i, 0)),
      ],
      out_specs=[
          pl.BlockSpec((B, D), lambda i: (i, 0)),
          pl.BlockSpec((B, 1), lambda i: (i, 0)),
      ],
      out_shape=[
          jax.ShapeDtypeStruct((N, D), jnp.float32),
          jax.ShapeDtypeStruct((N, 1), jnp.float32),
      ],
  )(h, deg0, deg1)


def _agg_body(scaled_hbm, src_hbm, dst_hbm, zeros2_hbm, part_hbm,
              acc, srcv, dstv, rows0, rows1, sem0, sem1, isem):
  c = lax.axis_index("c")
  s = lax.axis_index("s")
  w = c * NS + s
  nz = jnp.where(s == NS - 1, (LAST + ACCR - N) // 80, SLAB // 80)

  @pl.loop(0, 8)
  def _(i):
    @pl.when(i < nz)
    def _():
      pltpu.sync_copy(zeros2_hbm.at[pl.ds(0, 80)],
                      acc.at[pl.ds(s * SLAB + i * 80, 80)])
  pltpu.async_copy(src_hbm.at[pl.ds(w * EPP, EPP)], srcv, isem).wait()
  pltpu.async_copy(dst_hbm.at[w], dstv, isem).wait()
  plsc.subcore_barrier()

  pltpu.async_copy(scaled_hbm.at[srcv.at[pl.ds(0, C)]], rows0, sem0)

  @pl.loop(0, NCHUNK, step=2)
  def _(j):
    pltpu.make_async_copy(
        scaled_hbm.at[srcv.at[pl.ds(j * C, C)]], rows0, sem0).wait()
    pltpu.async_copy(
        scaled_hbm.at[srcv.at[pl.ds((j + 1) * C, C)]], rows1, sem1)
    pltpu.sync_copy(rows0, acc.at[dstv.at[j]], add=True)
    @pl.when(j + 2 < NCHUNK)
    def _():
      pltpu.async_copy(
          scaled_hbm.at[srcv.at[pl.ds((j + 2) * C, C)]], rows0, sem0)
    pltpu.make_async_copy(
        scaled_hbm.at[srcv.at[pl.ds((j + 1) * C, C)]], rows1, sem1).wait()
    pltpu.sync_copy(rows1, acc.at[dstv.at[j + 1]], add=True)

  plsc.subcore_barrier()
  @pl.when(s < NS - 1)
  def _():
    pltpu.sync_copy(acc.at[pl.ds(s * SLAB, SLAB)],
                    part_hbm.at[pl.ds(pl.multiple_of(c * N + s * SLAB, 8),
                                      SLAB)])

  @pl.when(s == NS - 1)
  def _():
    pltpu.sync_copy(acc.at[pl.ds(s * SLAB, LAST)],
                    part_hbm.at[pl.ds(pl.multiple_of(c * N + s * SLAB, 8),
                                      LAST)])


def _edge_aggregate(scaled, src_flat, dst3, zeros2):
  return pl.kernel(
      _agg_body,
      out_type=jax.ShapeDtypeStruct((NC * N, D), jnp.float32),
      mesh=_vmesh(),
      scratch_types=[
          pltpu.VMEM_SHARED((ACCR, D), jnp.float32),
          pltpu.VMEM((EPP,), jnp.int32),
          pltpu.VMEM((NCHUNK, C), jnp.int32),
          pltpu.VMEM((C, D), jnp.float32),
          pltpu.VMEM((C, D), jnp.float32),
          pltpu.SemaphoreType.DMA,
          pltpu.SemaphoreType.DMA,
          pltpu.SemaphoreType.DMA,
      ],
  )(scaled, src_flat, dst3, zeros2)


def _ln_body(p0_ref, p1_ref, sc_ref, dis_ref, x_ref, b_ref, g_ref, bt_ref,
             out_ref):
  agg = (p0_ref[...] + p1_ref[...] + sc_ref[...]) * dis_ref[...] + b_ref[...]
  mu = jnp.mean(agg, axis=-1, keepdims=True)
  zc = agg - mu
  var = jnp.mean(zc * zc, axis=-1, keepdims=True)
  ln = zc / jnp.sqrt(var + 1e-5) * g_ref[...] + bt_ref[...]
  out_ref[...] = jnp.maximum(ln, 0.0) + x_ref[...]


def _ln_residual(parts, scaled, dis, x, b, g, bt):
  B = 2000
  grid = (N // B,)
  row = lambda i: (i, 0)
  return pl.pallas_call(
      _ln_body,
      grid=grid,
      in_specs=[
          pl.BlockSpec((B, D), row),
          pl.BlockSpec((B, D), lambda i: (N // B + i, 0)),
          pl.BlockSpec((B, D), row),
          pl.BlockSpec((B, 1), row),
          pl.BlockSpec((B, D), row),
          pl.BlockSpec((1, D), lambda i: (0, 0)),
          pl.BlockSpec((1, D), lambda i: (0, 0)),
          pl.BlockSpec((1, D), lambda i: (0, 0)),
      ],
      out_specs=pl.BlockSpec((B, D), row),
      out_shape=jax.ShapeDtypeStruct((N, D), jnp.float32),
  )(parts, parts, scaled, dis, x, b, g, bt)


def kernel(x, edge_index, W, b, ln_gamma, ln_beta):
  ei = edge_index.astype(jnp.int32)
  npad = EPAD - E
  pad_i = jnp.arange(npad, dtype=jnp.int32)
  src_flat = jnp.concatenate([ei[0], pad_i % N])
  dst_flat = jnp.concatenate([ei[1], N + pad_i % (ACCR - N)])
  dst3 = dst_flat.reshape(NW, NCHUNK, C)
  zeros1 = jnp.zeros((NPAD,), jnp.float32)
  zeros2 = jnp.zeros((ZR, D), jnp.float32)

  h = _matmul(x, W)
  degp = _deg_partials(dst3, zeros1)
  deg0 = degp[0, :N].reshape(N, 1)
  deg1 = degp[1, :N].reshape(N, 1)

  scaled, dis = _row_scale(h, deg0, deg1)

  parts = _edge_aggregate(scaled, src_flat, dst3, zeros2)

  return _ln_residual(parts, scaled, dis, x, b.reshape(1, D),
                      ln_gamma.reshape(1, D), ln_beta.reshape(1, D))
